# baseline (speedup 1.0000x reference)
import jax
import jax.numpy as jnp
from jax.experimental import pallas as pl
from jax.experimental.pallas import tpu as pltpu

_CH = 128
_NBUF = 8


def _make_body(B, C, P):
    J = P // _CH
    N = B * J

    def body(x_ref, pe_ref, o_ref, inbuf, outbuf, insem, outsem):
        def in_copy(m, slot):
            b = m // J
            j = m % J
            return pltpu.make_async_copy(
                x_ref.at[b, :, pl.ds(j * _CH, _CH)],
                inbuf.at[slot],
                insem.at[slot],
            )

        def out_copy(m, slot):
            b = m // J
            j = m % J
            return pltpu.make_async_copy(
                outbuf.at[slot],
                o_ref.at[b, pl.ds(j * _CH, _CH), :],
                outsem.at[slot],
            )

        n = pl.program_id(0)
        slot = jax.lax.rem(n, _NBUF)

        @pl.when(n == 0)
        def _():
            for k in range(_NBUF - 1):
                in_copy(k, k).start()

        @pl.when(n >= _NBUF)
        def _():
            out_copy(n - _NBUF, slot).wait()

        in_copy(n, slot).wait()
        j = jax.lax.rem(n, J)
        outbuf[slot] = inbuf[slot].T + pe_ref[pl.ds(j * _CH, _CH), :]
        out_copy(n, slot).start()

        @pl.when(n + _NBUF - 1 < N)
        def _():
            in_copy(n + _NBUF - 1, jax.lax.rem(n + _NBUF - 1, _NBUF)).start()

        @pl.when(n == N - 1)
        def _():
            for m in range(N - _NBUF, N):
                out_copy(m, m % _NBUF).wait()

    return body, N


def kernel(image_feature, pe_table):
    B, C, H, W = image_feature.shape
    P = H * W
    x = image_feature.reshape(B, C, P)
    body, N = _make_body(B, C, P)
    return pl.pallas_call(
        body,
        grid=(N,),
        in_specs=[
            pl.BlockSpec(memory_space=pltpu.MemorySpace.HBM),
            pl.BlockSpec(memory_space=pltpu.MemorySpace.VMEM),
        ],
        out_specs=pl.BlockSpec(memory_space=pltpu.MemorySpace.HBM),
        out_shape=jax.ShapeDtypeStruct((B, P, C), image_feature.dtype),
        scratch_shapes=[
            pltpu.VMEM((_NBUF, C, _CH), jnp.float32),
            pltpu.VMEM((_NBUF, _CH, C), jnp.float32),
            pltpu.SemaphoreType.DMA((_NBUF,)),
            pltpu.SemaphoreType.DMA((_NBUF,)),
        ],
    )(x, pe_table)


# speedup vs baseline: 1.0412x; 1.0412x over previous
import jax
import jax.numpy as jnp
from jax.experimental import pallas as pl
from jax.experimental.pallas import tpu as pltpu

_CH = 1024
_NBUF = 8


def _make_body(B, C, P):
    J = P // _CH
    N = B * J

    def body(x_ref, pe_ref, o_ref, inbuf, outbuf, insem, outsem):
        def in_copy(m, slot):
            b = m // J
            j = m % J
            return pltpu.make_async_copy(
                x_ref.at[b, :, pl.ds(j * _CH, _CH)],
                inbuf.at[slot],
                insem.at[slot],
            )

        def out_copy(m, slot):
            b = m // J
            j = m % J
            return pltpu.make_async_copy(
                outbuf.at[slot],
                o_ref.at[b, pl.ds(j * _CH, _CH), :],
                outsem.at[slot],
            )

        n = pl.program_id(0)
        slot = jax.lax.rem(n, _NBUF)

        @pl.when(n == 0)
        def _():
            for k in range(_NBUF - 1):
                in_copy(k, k).start()

        @pl.when(n >= _NBUF)
        def _():
            out_copy(n - _NBUF, slot).wait()

        in_copy(n, slot).wait()
        j = jax.lax.rem(n, J)
        outbuf[slot] = inbuf[slot].T + pe_ref[pl.ds(j * _CH, _CH), :]
        out_copy(n, slot).start()

        @pl.when(n + _NBUF - 1 < N)
        def _():
            in_copy(n + _NBUF - 1, jax.lax.rem(n + _NBUF - 1, _NBUF)).start()

        @pl.when(n == N - 1)
        def _():
            for m in range(N - _NBUF, N):
                out_copy(m, m % _NBUF).wait()

    return body, N


def kernel(image_feature, pe_table):
    B, C, H, W = image_feature.shape
    P = H * W
    x = image_feature.reshape(B, C, P)
    body, N = _make_body(B, C, P)
    return pl.pallas_call(
        body,
        grid=(N,),
        in_specs=[
            pl.BlockSpec(memory_space=pltpu.MemorySpace.HBM),
            pl.BlockSpec(memory_space=pltpu.MemorySpace.VMEM),
        ],
        out_specs=pl.BlockSpec(memory_space=pltpu.MemorySpace.HBM),
        out_shape=jax.ShapeDtypeStruct((B, P, C), image_feature.dtype),
        scratch_shapes=[
            pltpu.VMEM((_NBUF, C, _CH), jnp.float32),
            pltpu.VMEM((_NBUF, _CH, C), jnp.float32),
            pltpu.SemaphoreType.DMA((_NBUF,)),
            pltpu.SemaphoreType.DMA((_NBUF,)),
        ],
    )(x, pe_table)
